# baseline (speedup 1.0000x reference)
import jax
import jax.numpy as jnp
from jax.experimental import pallas as pl
from jax.experimental.pallas import tpu as pltpu

_T = 32
_S = 512
_D = 128
_H = 128
_L = 64
_B = 8


def _gates(z, c):
    ti = jnp.tanh(z[:, 0:_H])
    tf = jnp.tanh(z[:, _H:2 * _H])
    tg = jnp.tanh(z[:, 2 * _H:3 * _H])
    to = jnp.tanh(z[:, 3 * _H:4 * _H])
    c_new = 0.5 * ((tf + 1.0) * c + (ti + 1.0) * tg)
    h_new = (0.5 * (to + 1.0)) * jnp.tanh(c_new)
    return h_new, c_new


def _dot(a, w):
    return jnp.dot(a, w, preferred_element_type=jnp.float32)


def _ld(ref, t):
    return ref[pl.ds(t, 1)][0]


def _sent_kernel(x_ref, len_ref,
                 wx0f, wh0f, b0f, wx0b, wh0b, b0b,
                 wx1f, wh1f, b1f, wx1b, wh1b, b1b,
                 enc_ref, ys0_ref):
    ln = len_ref[...]

    def bilayer(read_x, wxf, whf, bf, wxb, whb, bb, write_ys):
        wxfv, whfv, bfv = wxf[...], whf[...], bf[...]
        wxbv, whbv, bbv = wxb[...], whb[...], bb[...]

        def step(k, carry):
            hf, cf, hb, cb = carry
            t2 = _T - 1 - k
            m = ln > k.astype(jnp.float32)
            m2 = ln > t2.astype(jnp.float32)
            zf = _dot(read_x(k), wxfv) + _dot(hf, whfv) + bfv
            zb = _dot(read_x(t2), wxbv) + _dot(hb, whbv) + bbv
            hn, cn = _gates(zf, cf)
            hf = jnp.where(m, hn, hf)
            cf = jnp.where(m, cn, cf)
            hn2, cn2 = _gates(zb, cb)
            hb = jnp.where(m2, hn2, hb)
            cb = jnp.where(m2, cn2, cb)
            write_ys(k, hf, t2, hb)
            return hf, cf, hb, cb

        z = jnp.zeros((_S, _H), jnp.float32)
        return jax.lax.fori_loop(0, _T, step, (z, z, z, z), unroll=4)

    def write0(k, ysf, t2, ysb):
        ys0_ref[pl.ds(k, 1), :, 0:_H] = ysf.astype(jnp.bfloat16)[None]
        ys0_ref[pl.ds(t2, 1), :, _H:2 * _H] = ysb.astype(jnp.bfloat16)[None]

    h0f, _, h0b, _ = bilayer(lambda t: _ld(x_ref, t),
                             wx0f, wh0f, b0f, wx0b, wh0b, b0b, write0)
    h1f, _, h1b, _ = bilayer(lambda t: _ld(ys0_ref, t),
                             wx1f, wh1f, b1f, wx1b, wh1b, b1b,
                             lambda *_a: None)

    enc_ref[:, 0:_H] = h0f
    enc_ref[:, _H:2 * _H] = h0b
    enc_ref[:, 2 * _H:3 * _H] = h1f
    enc_ref[:, 3 * _H:4 * _H] = h1b


def _doc_kernel(dx_ref,
                wx0f, wh0f, b0f, wx0b, wh0b, b0b,
                wx1f, wh1f, b1f, wx1b, wh1b, b1b,
                wh, bh, out_ref, p0f_ref, p0b_ref, p1f_ref, p1b_ref,
                ys0_ref, ys1_ref):
    dxf = dx_ref[...].reshape(_L * _B, 4 * _H)
    p0f_ref[...] = (_dot(dxf, wx0f[...]) + b0f[...]).reshape(_L, _B, 4 * _H)
    p0b_ref[...] = (_dot(dxf, wx0b[...]) + b0b[...]).reshape(_L, _B, 4 * _H)

    def bilayer(pf_ref, pb_ref, whf, whb, ys_ref):
        whfv, whbv = whf[...], whb[...]

        def step(k, carry):
            hf, cf, hb, cb = carry
            t2 = _L - 1 - k
            zf = _ld(pf_ref, k) + _dot(hf, whfv)
            zb = _ld(pb_ref, t2) + _dot(hb, whbv)
            hf, cf = _gates(zf, cf)
            hb, cb = _gates(zb, cb)
            ys_ref[pl.ds(k, 1), :, 0:_H] = hf[None]
            ys_ref[pl.ds(t2, 1), :, _H:2 * _H] = hb[None]
            return hf, cf, hb, cb

        z = jnp.zeros((_B, _H), jnp.float32)
        jax.lax.fori_loop(0, _L, step, (z, z, z, z), unroll=8)

    bilayer(p0f_ref, p0b_ref, wh0f, wh0b, ys0_ref)

    ys0 = ys0_ref[...].reshape(_L * _B, 2 * _H)
    p1f_ref[...] = (_dot(ys0, wx1f[...]) + b1f[...]).reshape(_L, _B, 4 * _H)
    p1b_ref[...] = (_dot(ys0, wx1b[...]) + b1b[...]).reshape(_L, _B, 4 * _H)

    bilayer(p1f_ref, p1b_ref, wh1f, wh1b, ys1_ref)

    ys = ys1_ref[...].reshape(_L * _B, 2 * _H)
    out_ref[...] = _dot(ys, wh[...]) + bh[...]


def kernel(sent_emb, params, sent_lengths):
    p = params

    scale = jnp.concatenate([jnp.full((2 * _H,), 0.5, jnp.float32),
                             jnp.ones((_H,), jnp.float32),
                             jnp.full((_H,), 0.5, jnp.float32)])

    def w(prefix):
        return (p[prefix + 'Wi'].T * scale, p[prefix + 'Wh'].T * scale,
                p[prefix + 'b'][None] * scale)

    lens = sent_lengths.astype(jnp.float32)[:, None]
    xT = jnp.transpose(sent_emb, (1, 0, 2))

    sw = [x for pre in ('se0f_', 'se0b_', 'se1f_', 'se1b_') for x in w(pre)]
    dw = [x for pre in ('dl0f_', 'dl0b_', 'dl1f_', 'dl1b_') for x in w(pre)]

    sw = [a.astype(jnp.bfloat16) if i % 3 == 0 else a
          for i, a in enumerate(sw)]
    enc = pl.pallas_call(
        _sent_kernel,
        out_shape=jax.ShapeDtypeStruct((_S, 4 * _H), jnp.float32),
        scratch_shapes=[pltpu.VMEM((_T, _S, 2 * _H), jnp.bfloat16)],
    )(xT.astype(jnp.bfloat16), lens, *sw)

    dxT = enc.reshape(_B, _L, 4 * _H).transpose(1, 0, 2)

    logits = pl.pallas_call(
        _doc_kernel,
        out_shape=jax.ShapeDtypeStruct((_L * _B, 2), jnp.float32),
        scratch_shapes=[pltpu.VMEM((_L, _B, 4 * _H), jnp.float32)] * 4
        + [pltpu.VMEM((_L, _B, 2 * _H), jnp.float32)] * 2,
    )(dxT, *dw, p['h2s_W'].T, p['h2s_b'][None])

    out = logits.reshape(_L, _B, 2).transpose(1, 0, 2)
    return out[:, :_L - 1].reshape((_L - 1) * _B, 2)


# speedup vs baseline: 1.0289x; 1.0289x over previous
import jax
import jax.numpy as jnp
from jax.experimental import pallas as pl
from jax.experimental.pallas import tpu as pltpu

_T = 32
_S = 512
_D = 128
_H = 128
_L = 64
_B = 8


def _gates(z, c):
    ti = jnp.tanh(z[:, 0:_H])
    tf = jnp.tanh(z[:, _H:2 * _H])
    tg = jnp.tanh(z[:, 2 * _H:3 * _H])
    to = jnp.tanh(z[:, 3 * _H:4 * _H])
    c_new = 0.5 * ((tf + 1.0) * c + (ti + 1.0) * tg)
    h_new = (0.5 * (to + 1.0)) * jnp.tanh(c_new)
    return h_new, c_new


def _dot(a, w):
    return jnp.dot(a, w, preferred_element_type=jnp.float32)


def _ld(ref, t):
    return ref[pl.ds(t, 1)][0]


def _sent_kernel(x_ref, len_ref,
                 wx0f, wh0f, b0f, wx0b, wh0b, b0b,
                 wx1f, wh1f, b1f, wx1b, wh1b, b1b,
                 enc_ref, ys0_ref):
    ln = len_ref[...]

    def bilayer(read_x, wxf, whf, bf, wxb, whb, bb, write_ys):
        wxfv, whfv, bfv = wxf[...], whf[...], bf[...]
        wxbv, whbv, bbv = wxb[...], whb[...], bb[...]

        def step(k, carry):
            hf, cf, hb, cb = carry
            t2 = _T - 1 - k
            m = ln > k.astype(jnp.float32)
            m2 = ln > t2.astype(jnp.float32)
            zf = _dot(read_x(k), wxfv) + _dot(hf, whfv) + bfv
            zb = _dot(read_x(t2), wxbv) + _dot(hb, whbv) + bbv
            hn, cn = _gates(zf, cf)
            hf = jnp.where(m, hn, hf)
            cf = jnp.where(m, cn, cf)
            hn2, cn2 = _gates(zb, cb)
            hb = jnp.where(m2, hn2, hb)
            cb = jnp.where(m2, cn2, cb)
            write_ys(k, hf, t2, hb)
            return hf, cf, hb, cb

        z = jnp.zeros((_S, _H), jnp.float32)
        return jax.lax.fori_loop(0, _T, step, (z, z, z, z), unroll=4)

    def write0(k, ysf, t2, ysb):
        ys0_ref[pl.ds(k, 1), :, 0:_H] = ysf[None]
        ys0_ref[pl.ds(t2, 1), :, _H:2 * _H] = ysb[None]

    h0f, _, h0b, _ = bilayer(lambda t: _ld(x_ref, t),
                             wx0f, wh0f, b0f, wx0b, wh0b, b0b, write0)
    h1f, _, h1b, _ = bilayer(lambda t: _ld(ys0_ref, t),
                             wx1f, wh1f, b1f, wx1b, wh1b, b1b,
                             lambda *_a: None)

    enc_ref[:, 0:_H] = h0f
    enc_ref[:, _H:2 * _H] = h0b
    enc_ref[:, 2 * _H:3 * _H] = h1f
    enc_ref[:, 3 * _H:4 * _H] = h1b


def _doc_kernel(dx_ref,
                wx0f, wh0f, b0f, wx0b, wh0b, b0b,
                wx1f, wh1f, b1f, wx1b, wh1b, b1b,
                wh, bh, out_ref, p0f_ref, p0b_ref, p1f_ref, p1b_ref,
                ys0_ref, ys1_ref):
    dxf = dx_ref[...].reshape(_L * _B, 4 * _H)
    p0f_ref[...] = (_dot(dxf, wx0f[...]) + b0f[...]).reshape(_L, _B, 4 * _H)
    p0b_ref[...] = (_dot(dxf, wx0b[...]) + b0b[...]).reshape(_L, _B, 4 * _H)

    def bilayer(pf_ref, pb_ref, whf, whb, ys_ref):
        whfv, whbv = whf[...], whb[...]

        def step(k, carry):
            h2, c2 = carry
            t2 = _L - 1 - k
            zf = _ld(pf_ref, k) + _dot(h2[0:_B], whfv)
            zb = _ld(pb_ref, t2) + _dot(h2[_B:], whbv)
            h2, c2 = _gates(jnp.concatenate([zf, zb], axis=0), c2)
            ys_ref[pl.ds(k, 1), :, 0:_H] = h2[0:_B][None]
            ys_ref[pl.ds(t2, 1), :, _H:2 * _H] = h2[_B:][None]
            return h2, c2

        z = jnp.zeros((2 * _B, _H), jnp.float32)
        jax.lax.fori_loop(0, _L, step, (z, z), unroll=8)

    bilayer(p0f_ref, p0b_ref, wh0f, wh0b, ys0_ref)

    ys0 = ys0_ref[...].reshape(_L * _B, 2 * _H)
    p1f_ref[...] = (_dot(ys0, wx1f[...]) + b1f[...]).reshape(_L, _B, 4 * _H)
    p1b_ref[...] = (_dot(ys0, wx1b[...]) + b1b[...]).reshape(_L, _B, 4 * _H)

    bilayer(p1f_ref, p1b_ref, wh1f, wh1b, ys1_ref)

    ys = ys1_ref[...].reshape(_L * _B, 2 * _H)
    out_ref[...] = _dot(ys, wh[...]) + bh[...]


def kernel(sent_emb, params, sent_lengths):
    p = params

    scale = jnp.concatenate([jnp.full((2 * _H,), 0.5, jnp.float32),
                             jnp.ones((_H,), jnp.float32),
                             jnp.full((_H,), 0.5, jnp.float32)])

    def w(prefix):
        return (p[prefix + 'Wi'].T * scale, p[prefix + 'Wh'].T * scale,
                p[prefix + 'b'][None] * scale)

    lens = sent_lengths.astype(jnp.float32)[:, None]
    xT = jnp.transpose(sent_emb, (1, 0, 2))

    sw = [x for pre in ('se0f_', 'se0b_', 'se1f_', 'se1b_') for x in w(pre)]
    dw = [x for pre in ('dl0f_', 'dl0b_', 'dl1f_', 'dl1b_') for x in w(pre)]

    enc = pl.pallas_call(
        _sent_kernel,
        out_shape=jax.ShapeDtypeStruct((_S, 4 * _H), jnp.float32),
        scratch_shapes=[pltpu.VMEM((_T, _S, 2 * _H), jnp.float32)],
    )(xT, lens, *sw)

    dxT = enc.reshape(_B, _L, 4 * _H).transpose(1, 0, 2)

    logits = pl.pallas_call(
        _doc_kernel,
        out_shape=jax.ShapeDtypeStruct((_L * _B, 2), jnp.float32),
        scratch_shapes=[pltpu.VMEM((_L, _B, 4 * _H), jnp.float32)] * 4
        + [pltpu.VMEM((_L, _B, 2 * _H), jnp.float32)] * 2,
    )(dxT, *dw, p['h2s_W'].T, p['h2s_b'][None])

    out = logits.reshape(_L, _B, 2).transpose(1, 0, 2)
    return out[:, :_L - 1].reshape((_L - 1) * _B, 2)


# speedup vs baseline: 1.0411x; 1.0119x over previous
import jax
import jax.numpy as jnp
from jax.experimental import pallas as pl
from jax.experimental.pallas import tpu as pltpu

_T = 32
_S = 512
_D = 128
_H = 128
_L = 64
_B = 8


def _gates(z, c):
    ti = jnp.tanh(z[:, 0:_H])
    tf = jnp.tanh(z[:, _H:2 * _H])
    tg = jnp.tanh(z[:, 2 * _H:3 * _H])
    to = jnp.tanh(z[:, 3 * _H:4 * _H])
    c_new = 0.5 * ((tf + 1.0) * c + (ti + 1.0) * tg)
    h_new = (0.5 * (to + 1.0)) * jnp.tanh(c_new)
    return h_new, c_new


def _dot(a, w):
    return jnp.dot(a, w, preferred_element_type=jnp.float32)


def _ld(ref, t):
    return ref[pl.ds(t, 1)][0]


def _sent_kernel(x_ref, len_ref,
                 wx0f, wh0f, b0f, wx0b, wh0b, b0b,
                 wx1f, wh1f, b1f, wx1b, wh1b, b1b,
                 enc_ref, ys0_ref):
    ln = len_ref[...]

    def bilayer(read_x, wxf, whf, bf, wxb, whb, bb, write_ys):
        wxfv, whfv, bfv = wxf[...], whf[...], bf[...]
        wxbv, whbv, bbv = wxb[...], whb[...], bb[...]

        def step(k, carry):
            hf, cf, hb, cb = carry
            t2 = _T - 1 - k
            m = ln > k.astype(jnp.float32)
            m2 = ln > t2.astype(jnp.float32)
            zf = _dot(read_x(k), wxfv) + _dot(hf, whfv) + bfv
            zb = _dot(read_x(t2), wxbv) + _dot(hb, whbv) + bbv
            hn, cn = _gates(zf, cf)
            hf = jnp.where(m, hn, hf)
            cf = jnp.where(m, cn, cf)
            hn2, cn2 = _gates(zb, cb)
            hb = jnp.where(m2, hn2, hb)
            cb = jnp.where(m2, cn2, cb)
            write_ys(k, hf, t2, hb)
            return hf, cf, hb, cb

        z = jnp.zeros((_S, _H), jnp.float32)
        return jax.lax.fori_loop(0, _T, step, (z, z, z, z), unroll=8)

    def write0(k, ysf, t2, ysb):
        ys0_ref[pl.ds(k, 1), :, 0:_H] = ysf[None]
        ys0_ref[pl.ds(t2, 1), :, _H:2 * _H] = ysb[None]

    h0f, _, h0b, _ = bilayer(lambda t: _ld(x_ref, t),
                             wx0f, wh0f, b0f, wx0b, wh0b, b0b, write0)
    h1f, _, h1b, _ = bilayer(lambda t: _ld(ys0_ref, t),
                             wx1f, wh1f, b1f, wx1b, wh1b, b1b,
                             lambda *_a: None)

    enc_ref[:, 0:_H] = h0f
    enc_ref[:, _H:2 * _H] = h0b
    enc_ref[:, 2 * _H:3 * _H] = h1f
    enc_ref[:, 3 * _H:4 * _H] = h1b


def _doc_kernel(dx_ref,
                wx0f, wh0f, b0f, wx0b, wh0b, b0b,
                wx1f, wh1f, b1f, wx1b, wh1b, b1b,
                wh, bh, out_ref, p0f_ref, p0b_ref, p1f_ref, p1b_ref,
                ys0_ref, ys1_ref):
    dxf = dx_ref[...].reshape(_L * _B, 4 * _H)
    p0f_ref[...] = (_dot(dxf, wx0f[...]) + b0f[...]).reshape(_L, _B, 4 * _H)
    p0b_ref[...] = (_dot(dxf, wx0b[...]) + b0b[...]).reshape(_L, _B, 4 * _H)

    def bilayer(pf_ref, pb_ref, whf, whb, ys_ref):
        whfv, whbv = whf[...], whb[...]

        def step(k, carry):
            h2, c2 = carry
            t2 = _L - 1 - k
            zf = _ld(pf_ref, k) + _dot(h2[0:_B], whfv)
            zb = _ld(pb_ref, t2) + _dot(h2[_B:], whbv)
            h2, c2 = _gates(jnp.concatenate([zf, zb], axis=0), c2)
            ys_ref[pl.ds(k, 1), :, 0:_H] = h2[0:_B][None]
            ys_ref[pl.ds(t2, 1), :, _H:2 * _H] = h2[_B:][None]
            return h2, c2

        z = jnp.zeros((2 * _B, _H), jnp.float32)
        jax.lax.fori_loop(0, _L, step, (z, z), unroll=8)

    bilayer(p0f_ref, p0b_ref, wh0f, wh0b, ys0_ref)

    ys0 = ys0_ref[...].reshape(_L * _B, 2 * _H)
    p1f_ref[...] = (_dot(ys0, wx1f[...]) + b1f[...]).reshape(_L, _B, 4 * _H)
    p1b_ref[...] = (_dot(ys0, wx1b[...]) + b1b[...]).reshape(_L, _B, 4 * _H)

    bilayer(p1f_ref, p1b_ref, wh1f, wh1b, ys1_ref)

    ys = ys1_ref[...].reshape(_L * _B, 2 * _H)
    out_ref[...] = _dot(ys, wh[...]) + bh[...]


def kernel(sent_emb, params, sent_lengths):
    p = params

    scale = jnp.concatenate([jnp.full((2 * _H,), 0.5, jnp.float32),
                             jnp.ones((_H,), jnp.float32),
                             jnp.full((_H,), 0.5, jnp.float32)])

    def w(prefix):
        return (p[prefix + 'Wi'].T * scale, p[prefix + 'Wh'].T * scale,
                p[prefix + 'b'][None] * scale)

    lens = sent_lengths.astype(jnp.float32)[:, None]
    xT = jnp.transpose(sent_emb, (1, 0, 2))

    sw = [x for pre in ('se0f_', 'se0b_', 'se1f_', 'se1b_') for x in w(pre)]
    dw = [x for pre in ('dl0f_', 'dl0b_', 'dl1f_', 'dl1b_') for x in w(pre)]

    enc = pl.pallas_call(
        _sent_kernel,
        out_shape=jax.ShapeDtypeStruct((_S, 4 * _H), jnp.float32),
        scratch_shapes=[pltpu.VMEM((_T, _S, 2 * _H), jnp.float32)],
    )(xT, lens, *sw)

    dxT = enc.reshape(_B, _L, 4 * _H).transpose(1, 0, 2)

    logits = pl.pallas_call(
        _doc_kernel,
        out_shape=jax.ShapeDtypeStruct((_L * _B, 2), jnp.float32),
        scratch_shapes=[pltpu.VMEM((_L, _B, 4 * _H), jnp.float32)] * 4
        + [pltpu.VMEM((_L, _B, 2 * _H), jnp.float32)] * 2,
    )(dxT, *dw, p['h2s_W'].T, p['h2s_b'][None])

    out = logits.reshape(_L, _B, 2).transpose(1, 0, 2)
    return out[:, :_L - 1].reshape((_L - 1) * _B, 2)


# speedup vs baseline: 1.1574x; 1.1117x over previous
import jax
import jax.numpy as jnp
from jax.experimental import pallas as pl
from jax.experimental.pallas import tpu as pltpu

_T = 32
_S = 512
_D = 128
_H = 128
_L = 64
_B = 8


def _gates(z, c):
    ti = jnp.tanh(z[:, 0:_H])
    tf = jnp.tanh(z[:, _H:2 * _H])
    tg = jnp.tanh(z[:, 2 * _H:3 * _H])
    to = jnp.tanh(z[:, 3 * _H:4 * _H])
    c_new = 0.5 * ((tf + 1.0) * c + (ti + 1.0) * tg)
    h_new = (0.5 * (to + 1.0)) * jnp.tanh(c_new)
    return h_new, c_new


def _dot(a, w):
    return jnp.dot(a, w, preferred_element_type=jnp.float32)


def _ld(ref, t):
    return ref[pl.ds(t, 1)][0]


def _sent_kernel(x_ref, len_ref,
                 w0f, b0f, w0b, b0b,
                 w1f, b1f, w1b, b1b,
                 enc_ref, ys0_ref):
    ln = len_ref[...]

    def bilayer(read_x, wf, bf, wb, bb, write_ys):
        wfv, bfv = wf[...], bf[...]
        wbv, bbv = wb[...], bb[...]

        def step(k, carry):
            hf, cf, hb, cb = carry
            t2 = _T - 1 - k
            m = ln > k.astype(jnp.float32)
            m2 = ln > t2.astype(jnp.float32)
            zf = _dot(jnp.concatenate([read_x(k), hf], axis=1), wfv) + bfv
            zb = _dot(jnp.concatenate([read_x(t2), hb], axis=1), wbv) + bbv
            hn, cn = _gates(zf, cf)
            hf = jnp.where(m, hn, hf)
            cf = jnp.where(m, cn, cf)
            hn2, cn2 = _gates(zb, cb)
            hb = jnp.where(m2, hn2, hb)
            cb = jnp.where(m2, cn2, cb)
            write_ys(k, hf, t2, hb)
            return hf, cf, hb, cb

        z = jnp.zeros((_S, _H), jnp.float32)
        return jax.lax.fori_loop(0, _T, step, (z, z, z, z), unroll=8)

    def write0(k, ysf, t2, ysb):
        ys0_ref[pl.ds(k, 1), :, 0:_H] = ysf[None]
        ys0_ref[pl.ds(t2, 1), :, _H:2 * _H] = ysb[None]

    h0f, _, h0b, _ = bilayer(lambda t: _ld(x_ref, t),
                             w0f, b0f, w0b, b0b, write0)
    h1f, _, h1b, _ = bilayer(lambda t: _ld(ys0_ref, t),
                             w1f, b1f, w1b, b1b,
                             lambda *_a: None)

    enc_ref[:, 0:_H] = h0f
    enc_ref[:, _H:2 * _H] = h0b
    enc_ref[:, 2 * _H:3 * _H] = h1f
    enc_ref[:, 3 * _H:4 * _H] = h1b


def _doc_kernel(dx_ref,
                wx0f, wh0f, b0f, wx0b, wh0b, b0b,
                wx1f, wh1f, b1f, wx1b, wh1b, b1b,
                wh, bh, out_ref, p0f_ref, p0b_ref, p1f_ref, p1b_ref,
                ys0_ref, ys1_ref):
    dxf = dx_ref[...].reshape(_L * _B, 4 * _H)
    p0f_ref[...] = (_dot(dxf, wx0f[...]) + b0f[...]).reshape(_L, _B, 4 * _H)
    p0b_ref[...] = (_dot(dxf, wx0b[...]) + b0b[...]).reshape(_L, _B, 4 * _H)

    def bilayer(pf_ref, pb_ref, whf, whb, ys_ref):
        whfv, whbv = whf[...], whb[...]

        def step(k, carry):
            h2, c2 = carry
            t2 = _L - 1 - k
            zf = _ld(pf_ref, k) + _dot(h2[0:_B], whfv)
            zb = _ld(pb_ref, t2) + _dot(h2[_B:], whbv)
            h2, c2 = _gates(jnp.concatenate([zf, zb], axis=0), c2)
            ys_ref[pl.ds(k, 1), :, 0:_H] = h2[0:_B][None]
            ys_ref[pl.ds(t2, 1), :, _H:2 * _H] = h2[_B:][None]
            return h2, c2

        z = jnp.zeros((2 * _B, _H), jnp.float32)
        jax.lax.fori_loop(0, _L, step, (z, z), unroll=8)

    bilayer(p0f_ref, p0b_ref, wh0f, wh0b, ys0_ref)

    ys0 = ys0_ref[...].reshape(_L * _B, 2 * _H)
    p1f_ref[...] = (_dot(ys0, wx1f[...]) + b1f[...]).reshape(_L, _B, 4 * _H)
    p1b_ref[...] = (_dot(ys0, wx1b[...]) + b1b[...]).reshape(_L, _B, 4 * _H)

    bilayer(p1f_ref, p1b_ref, wh1f, wh1b, ys1_ref)

    ys = ys1_ref[...].reshape(_L * _B, 2 * _H)
    out_ref[...] = _dot(ys, wh[...]) + bh[...]


def kernel(sent_emb, params, sent_lengths):
    p = params

    scale = jnp.concatenate([jnp.full((2 * _H,), 0.5, jnp.float32),
                             jnp.ones((_H,), jnp.float32),
                             jnp.full((_H,), 0.5, jnp.float32)])

    def w(prefix):
        return (p[prefix + 'Wi'].T * scale, p[prefix + 'Wh'].T * scale,
                p[prefix + 'b'][None] * scale)

    def wcat(prefix):
        wx, wh, b = w(prefix)
        return jnp.concatenate([wx, wh], axis=0), b

    lens = sent_lengths.astype(jnp.float32)[:, None]
    xT = jnp.transpose(sent_emb, (1, 0, 2))

    sw = [x for pre in ('se0f_', 'se0b_', 'se1f_', 'se1b_')
          for x in wcat(pre)]
    dw = [x for pre in ('dl0f_', 'dl0b_', 'dl1f_', 'dl1b_') for x in w(pre)]

    enc = pl.pallas_call(
        _sent_kernel,
        out_shape=jax.ShapeDtypeStruct((_S, 4 * _H), jnp.float32),
        scratch_shapes=[pltpu.VMEM((_T, _S, 2 * _H), jnp.float32)],
    )(xT, lens, *sw)

    dxT = enc.reshape(_B, _L, 4 * _H).transpose(1, 0, 2)

    logits = pl.pallas_call(
        _doc_kernel,
        out_shape=jax.ShapeDtypeStruct((_L * _B, 2), jnp.float32),
        scratch_shapes=[pltpu.VMEM((_L, _B, 4 * _H), jnp.float32)] * 4
        + [pltpu.VMEM((_L, _B, 2 * _H), jnp.float32)] * 2,
    )(dxT, *dw, p['h2s_W'].T, p['h2s_b'][None])

    out = logits.reshape(_L, _B, 2).transpose(1, 0, 2)
    return out[:, :_L - 1].reshape((_L - 1) * _B, 2)


# speedup vs baseline: 1.1948x; 1.0323x over previous
import jax
import jax.numpy as jnp
from jax.experimental import pallas as pl
from jax.experimental.pallas import tpu as pltpu

_T = 32
_S = 512
_D = 128
_H = 128
_L = 64
_B = 8


def _gates(z, c):
    ti = jnp.tanh(z[:, 0:_H])
    tf = jnp.tanh(z[:, _H:2 * _H])
    tg = jnp.tanh(z[:, 2 * _H:3 * _H])
    to = jnp.tanh(z[:, 3 * _H:4 * _H])
    c_new = 0.5 * ((tf + 1.0) * c + (ti + 1.0) * tg)
    h_new = (0.5 * (to + 1.0)) * jnp.tanh(c_new)
    return h_new, c_new


def _dot(a, w):
    return jnp.dot(a, w, preferred_element_type=jnp.float32)


def _ld(ref, t):
    return ref[pl.ds(t, 1)][0]


def _sent_kernel(x_ref, len_ref,
                 w0f, b0f, w0b, b0b,
                 w1f, b1f, w1b, b1b,
                 enc_ref, ys0_ref):
    ln = len_ref[...]

    def bilayer(read_x, wf, bf, wb, bb, write_ys):
        wfv, bfv = wf[...], bf[...]
        wbv, bbv = wb[...], bb[...]

        def step(k, carry):
            hf, cf, hb, cb = carry
            t2 = _T - 1 - k
            m = ln > k.astype(jnp.float32)
            m2 = ln > t2.astype(jnp.float32)
            zf = _dot(jnp.concatenate([read_x(k), hf], axis=1), wfv) + bfv
            zb = _dot(jnp.concatenate([read_x(t2), hb], axis=1), wbv) + bbv
            hn, cn = _gates(zf, cf)
            hf = jnp.where(m, hn, hf)
            cf = jnp.where(m, cn, cf)
            hn2, cn2 = _gates(zb, cb)
            hb = jnp.where(m2, hn2, hb)
            cb = jnp.where(m2, cn2, cb)
            write_ys(k, hf, t2, hb)
            return hf, cf, hb, cb

        z = jnp.zeros((_S, _H), jnp.float32)
        return jax.lax.fori_loop(0, _T, step, (z, z, z, z), unroll=8)

    def write0(k, ysf, t2, ysb):
        ys0_ref[pl.ds(k, 1), :, 0:_H] = ysf[None]
        ys0_ref[pl.ds(t2, 1), :, _H:2 * _H] = ysb[None]

    h0f, _, h0b, _ = bilayer(lambda t: _ld(x_ref, t),
                             w0f, b0f, w0b, b0b, write0)
    h1f, _, h1b, _ = bilayer(lambda t: _ld(ys0_ref, t),
                             w1f, b1f, w1b, b1b,
                             lambda *_a: None)

    enc = jnp.concatenate([h0f, h0b, h1f, h1b], axis=1)
    enc_ref[...] = jnp.swapaxes(enc.reshape(_B, _L, 4 * _H), 0, 1)


def _doc_kernel(dx_ref,
                wx0f, wh0f, b0f, wx0b, wh0b, b0b,
                wx1f, wh1f, b1f, wx1b, wh1b, b1b,
                wh, bh, out_ref, p0f_ref, p0b_ref, p1f_ref, p1b_ref,
                ys0_ref, ys1_ref):
    dxf = dx_ref[...].reshape(_L * _B, 4 * _H)
    p0f_ref[...] = (_dot(dxf, wx0f[...]) + b0f[...]).reshape(_L, _B, 4 * _H)
    p0b_ref[...] = (_dot(dxf, wx0b[...]) + b0b[...]).reshape(_L, _B, 4 * _H)

    def bilayer(pf_ref, pb_ref, whf, whb, ys_ref):
        whfv, whbv = whf[...], whb[...]

        def step(k, carry):
            h2, c2 = carry
            t2 = _L - 1 - k
            zf = _ld(pf_ref, k) + _dot(h2[0:_B], whfv)
            zb = _ld(pb_ref, t2) + _dot(h2[_B:], whbv)
            h2, c2 = _gates(jnp.concatenate([zf, zb], axis=0), c2)
            ys_ref[pl.ds(k, 1), :, 0:_H] = h2[0:_B][None]
            ys_ref[pl.ds(t2, 1), :, _H:2 * _H] = h2[_B:][None]
            return h2, c2

        z = jnp.zeros((2 * _B, _H), jnp.float32)
        jax.lax.fori_loop(0, _L, step, (z, z), unroll=8)

    bilayer(p0f_ref, p0b_ref, wh0f, wh0b, ys0_ref)

    ys0 = ys0_ref[...].reshape(_L * _B, 2 * _H)
    p1f_ref[...] = (_dot(ys0, wx1f[...]) + b1f[...]).reshape(_L, _B, 4 * _H)
    p1b_ref[...] = (_dot(ys0, wx1b[...]) + b1b[...]).reshape(_L, _B, 4 * _H)

    bilayer(p1f_ref, p1b_ref, wh1f, wh1b, ys1_ref)

    ys = ys1_ref[...].reshape(_L * _B, 2 * _H)
    out_ref[...] = _dot(ys, wh[...]) + bh[...]


def kernel(sent_emb, params, sent_lengths):
    p = params

    scale = jnp.concatenate([jnp.full((2 * _H,), 0.5, jnp.float32),
                             jnp.ones((_H,), jnp.float32),
                             jnp.full((_H,), 0.5, jnp.float32)])

    def w(prefix):
        return (p[prefix + 'Wi'].T * scale, p[prefix + 'Wh'].T * scale,
                p[prefix + 'b'][None] * scale)

    def wcat(prefix):
        wx, wh, b = w(prefix)
        return jnp.concatenate([wx, wh], axis=0), b

    lens = sent_lengths.astype(jnp.float32)[:, None]
    xT = jnp.transpose(sent_emb, (1, 0, 2))

    sw = [x for pre in ('se0f_', 'se0b_', 'se1f_', 'se1b_')
          for x in wcat(pre)]
    dw = [x for pre in ('dl0f_', 'dl0b_', 'dl1f_', 'dl1b_') for x in w(pre)]

    dxT = pl.pallas_call(
        _sent_kernel,
        out_shape=jax.ShapeDtypeStruct((_L, _B, 4 * _H), jnp.float32),
        scratch_shapes=[pltpu.VMEM((_T, _S, 2 * _H), jnp.float32)],
    )(xT, lens, *sw)

    logits = pl.pallas_call(
        _doc_kernel,
        out_shape=jax.ShapeDtypeStruct((_L * _B, 2), jnp.float32),
        scratch_shapes=[pltpu.VMEM((_L, _B, 4 * _H), jnp.float32)] * 4
        + [pltpu.VMEM((_L, _B, 2 * _H), jnp.float32)] * 2,
    )(dxT, *dw, p['h2s_W'].T, p['h2s_b'][None])

    out = logits.reshape(_L, _B, 2).transpose(1, 0, 2)
    return out[:, :_L - 1].reshape((_L - 1) * _B, 2)


# speedup vs baseline: 1.2647x; 1.0585x over previous
import jax
import jax.numpy as jnp
from jax.experimental import pallas as pl
from jax.experimental.pallas import tpu as pltpu

_T = 32
_S = 512
_D = 128
_H = 128
_L = 64
_B = 8


def _gates(z, c):
    ti = jnp.tanh(z[:, 0:_H])
    tf = jnp.tanh(z[:, _H:2 * _H])
    tg = jnp.tanh(z[:, 2 * _H:3 * _H])
    to = jnp.tanh(z[:, 3 * _H:4 * _H])
    c_new = 0.5 * ((tf + 1.0) * c + (ti + 1.0) * tg)
    h_new = (0.5 * (to + 1.0)) * jnp.tanh(c_new)
    return h_new, c_new


def _dot(a, w):
    return jnp.dot(a, w, preferred_element_type=jnp.float32)


def _ld(ref, t):
    return ref[pl.ds(t, 1)][0]


def _sent_kernel(x_hbm, len_ref,
                 w0f, b0f, w0b, b0b,
                 w1f, b1f, w1b, b1b,
                 enc_ref, xt_ref, ys0_ref, sems):
    ln = len_ref[...]

    def dma(t):
        return pltpu.make_async_copy(x_hbm.at[:, t], xt_ref.at[t], sems.at[t])

    for t in range(_T):
        dma(t).start()

    w0fv, b0fv = w0f[...], b0f[...]
    w0bv, b0bv = w0b[...], b0b[...]

    def cell(z_pre, h, c, m):
        hn, cn = _gates(z_pre, c)
        return jnp.where(m, hn, h), jnp.where(m, cn, c)

    zz = jnp.zeros((_S, _H), jnp.float32)
    hf = cf = hb = cb = zz
    for k in range(_T):
        t2 = _T - 1 - k
        if k < _T // 2:
            dma(k).wait()
            dma(t2).wait()
        m = ln > jnp.float32(k)
        m2 = ln > jnp.float32(t2)
        zf = _dot(jnp.concatenate([xt_ref[k], hf], axis=1), w0fv) + b0fv
        zb = _dot(jnp.concatenate([xt_ref[t2], hb], axis=1), w0bv) + b0bv
        hf, cf = cell(zf, hf, cf, m)
        hb, cb = cell(zb, hb, cb, m2)
        ys0_ref[k, :, 0:_H] = hf
        ys0_ref[t2, :, _H:2 * _H] = hb
    h0f, h0b = hf, hb

    w1fv, b1fv = w1f[...], b1f[...]
    w1bv, b1bv = w1b[...], b1b[...]

    def step1(k, carry):
        hf, cf, hb, cb = carry
        t2 = _T - 1 - k
        m = ln > k.astype(jnp.float32)
        m2 = ln > t2.astype(jnp.float32)
        zf = _dot(jnp.concatenate([_ld(ys0_ref, k), hf], axis=1),
                  w1fv) + b1fv
        zb = _dot(jnp.concatenate([_ld(ys0_ref, t2), hb], axis=1),
                  w1bv) + b1bv
        hf, cf = cell(zf, hf, cf, m)
        hb, cb = cell(zb, hb, cb, m2)
        return hf, cf, hb, cb

    h1f, _, h1b, _ = jax.lax.fori_loop(0, _T, step1, (zz, zz, zz, zz),
                                       unroll=8)

    enc = jnp.concatenate([h0f, h0b, h1f, h1b], axis=1)
    enc_ref[...] = jnp.swapaxes(enc.reshape(_B, _L, 4 * _H), 0, 1)


def _doc_kernel(dx_ref,
                wx0f, wh0f, b0f, wx0b, wh0b, b0b,
                wx1f, wh1f, b1f, wx1b, wh1b, b1b,
                wh, bh, out_ref, p0f_ref, p0b_ref, p1f_ref, p1b_ref,
                ys0_ref, ys1_ref):
    dxf = dx_ref[...].reshape(_L * _B, 4 * _H)
    p0f_ref[...] = (_dot(dxf, wx0f[...]) + b0f[...]).reshape(_L, _B, 4 * _H)
    p0b_ref[...] = (_dot(dxf, wx0b[...]) + b0b[...]).reshape(_L, _B, 4 * _H)

    def bilayer(pf_ref, pb_ref, whf, whb, ys_ref):
        whfv, whbv = whf[...], whb[...]

        def step(k, carry):
            h2, c2 = carry
            t2 = _L - 1 - k
            zf = _ld(pf_ref, k) + _dot(h2[0:_B], whfv)
            zb = _ld(pb_ref, t2) + _dot(h2[_B:], whbv)
            h2, c2 = _gates(jnp.concatenate([zf, zb], axis=0), c2)
            ys_ref[pl.ds(k, 1), :, 0:_H] = h2[0:_B][None]
            ys_ref[pl.ds(t2, 1), :, _H:2 * _H] = h2[_B:][None]
            return h2, c2

        z = jnp.zeros((2 * _B, _H), jnp.float32)
        jax.lax.fori_loop(0, _L, step, (z, z), unroll=8)

    bilayer(p0f_ref, p0b_ref, wh0f, wh0b, ys0_ref)

    ys0 = ys0_ref[...].reshape(_L * _B, 2 * _H)
    p1f_ref[...] = (_dot(ys0, wx1f[...]) + b1f[...]).reshape(_L, _B, 4 * _H)
    p1b_ref[...] = (_dot(ys0, wx1b[...]) + b1b[...]).reshape(_L, _B, 4 * _H)

    bilayer(p1f_ref, p1b_ref, wh1f, wh1b, ys1_ref)

    ys = ys1_ref[...].reshape(_L * _B, 2 * _H)
    out_ref[...] = _dot(ys, wh[...]) + bh[...]


def kernel(sent_emb, params, sent_lengths):
    p = params

    scale = jnp.concatenate([jnp.full((2 * _H,), 0.5, jnp.float32),
                             jnp.ones((_H,), jnp.float32),
                             jnp.full((_H,), 0.5, jnp.float32)])

    def w(prefix):
        return (p[prefix + 'Wi'].T * scale, p[prefix + 'Wh'].T * scale,
                p[prefix + 'b'][None] * scale)

    def wcat(prefix):
        wx, wh, b = w(prefix)
        return jnp.concatenate([wx, wh], axis=0), b

    lens = sent_lengths.astype(jnp.float32)[:, None]

    sw = [x for pre in ('se0f_', 'se0b_', 'se1f_', 'se1b_')
          for x in wcat(pre)]
    dw = [x for pre in ('dl0f_', 'dl0b_', 'dl1f_', 'dl1b_') for x in w(pre)]

    vmem = pl.BlockSpec(memory_space=pltpu.MemorySpace.VMEM)
    dxT = pl.pallas_call(
        _sent_kernel,
        in_specs=[pl.BlockSpec(memory_space=pltpu.MemorySpace.HBM)]
        + [vmem] * (1 + len(sw)),
        out_shape=jax.ShapeDtypeStruct((_L, _B, 4 * _H), jnp.float32),
        scratch_shapes=[pltpu.VMEM((_T, _S, _D), jnp.float32),
                        pltpu.VMEM((_T, _S, 2 * _H), jnp.float32),
                        pltpu.SemaphoreType.DMA((_T,))],
    )(sent_emb, lens, *sw)

    logits = pl.pallas_call(
        _doc_kernel,
        out_shape=jax.ShapeDtypeStruct((_L * _B, 2), jnp.float32),
        scratch_shapes=[pltpu.VMEM((_L, _B, 4 * _H), jnp.float32)] * 4
        + [pltpu.VMEM((_L, _B, 2 * _H), jnp.float32)] * 2,
    )(dxT, *dw, p['h2s_W'].T, p['h2s_b'][None])

    out = logits.reshape(_L, _B, 2).transpose(1, 0, 2)
    return out[:, :_L - 1].reshape((_L - 1) * _B, 2)


# speedup vs baseline: 1.2771x; 1.0098x over previous
import jax
import jax.numpy as jnp
from jax.experimental import pallas as pl
from jax.experimental.pallas import tpu as pltpu

_T = 32
_S = 512
_D = 128
_H = 128
_L = 64
_B = 8


def _gates(z, c):
    ti = jnp.tanh(z[:, 0:_H])
    tf = jnp.tanh(z[:, _H:2 * _H])
    tg = jnp.tanh(z[:, 2 * _H:3 * _H])
    to = jnp.tanh(z[:, 3 * _H:4 * _H])
    c_new = 0.5 * ((tf + 1.0) * c + (ti + 1.0) * tg)
    h_new = (0.5 * (to + 1.0)) * jnp.tanh(c_new)
    return h_new, c_new


def _dot(a, w):
    return jnp.dot(a, w, preferred_element_type=jnp.float32)


def _ld(ref, t):
    return ref[pl.ds(t, 1)][0]


def _kernel(x_hbm, len_ref,
            w0f, b0f, w0b, b0b, w1f, b1f, w1b, b1b,
            dwx0f, dwh0f, db0f, dwx0b, dwh0b, db0b,
            dwx1f, dwh1f, db1f, dwx1b, dwh1b, db1b,
            wh, bh, out_ref,
            xt_ref, ys0_ref, p0f_ref, p0b_ref, p1f_ref, p1b_ref,
            ysd0_ref, ysd1_ref, sems):
    ln = len_ref[...]

    def dma(t):
        return pltpu.make_async_copy(x_hbm.at[:, t], xt_ref.at[t], sems.at[t])

    for t in range(_T):
        dma(t).start()

    w0fv, b0fv = w0f[...], b0f[...]
    w0bv, b0bv = w0b[...], b0b[...]

    def cell(z_pre, h, c, m):
        hn, cn = _gates(z_pre, c)
        return jnp.where(m, hn, h), jnp.where(m, cn, c)

    zz = jnp.zeros((_S, _H), jnp.float32)
    hf = cf = hb = cb = zz
    for k in range(_T):
        t2 = _T - 1 - k
        if k < _T // 2:
            dma(k).wait()
            dma(t2).wait()
        m = ln > jnp.float32(k)
        m2 = ln > jnp.float32(t2)
        zf = _dot(jnp.concatenate([xt_ref[k], hf], axis=1), w0fv) + b0fv
        zb = _dot(jnp.concatenate([xt_ref[t2], hb], axis=1), w0bv) + b0bv
        hf, cf = cell(zf, hf, cf, m)
        hb, cb = cell(zb, hb, cb, m2)
        ys0_ref[k, :, 0:_H] = hf
        ys0_ref[t2, :, _H:2 * _H] = hb
    h0f, h0b = hf, hb

    w1fv, b1fv = w1f[...], b1f[...]
    w1bv, b1bv = w1b[...], b1b[...]

    def step1(k, carry):
        hf, cf, hb, cb = carry
        t2 = _T - 1 - k
        m = ln > k.astype(jnp.float32)
        m2 = ln > t2.astype(jnp.float32)
        zf = _dot(jnp.concatenate([_ld(ys0_ref, k), hf], axis=1),
                  w1fv) + b1fv
        zb = _dot(jnp.concatenate([_ld(ys0_ref, t2), hb], axis=1),
                  w1bv) + b1bv
        hf, cf = cell(zf, hf, cf, m)
        hb, cb = cell(zb, hb, cb, m2)
        return hf, cf, hb, cb

    h1f, _, h1b, _ = jax.lax.fori_loop(0, _T, step1, (zz, zz, zz, zz),
                                       unroll=8)

    enc = jnp.concatenate([h0f, h0b, h1f, h1b], axis=1)
    dxf = jnp.swapaxes(enc.reshape(_B, _L, 4 * _H), 0, 1).reshape(
        _L * _B, 4 * _H)

    p0f_ref[...] = (_dot(dxf, dwx0f[...]) + db0f[...]).reshape(_L, _B, 4 * _H)
    p0b_ref[...] = (_dot(dxf, dwx0b[...]) + db0b[...]).reshape(_L, _B, 4 * _H)

    def bilayer(pf_ref, pb_ref, whf, whb, ys_ref):
        whfv, whbv = whf[...], whb[...]

        def step(k, carry):
            h2, c2 = carry
            t2 = _L - 1 - k
            zf = _ld(pf_ref, k) + _dot(h2[0:_B], whfv)
            zb = _ld(pb_ref, t2) + _dot(h2[_B:], whbv)
            h2, c2 = _gates(jnp.concatenate([zf, zb], axis=0), c2)
            ys_ref[pl.ds(k, 1), :, 0:_H] = h2[0:_B][None]
            ys_ref[pl.ds(t2, 1), :, _H:2 * _H] = h2[_B:][None]
            return h2, c2

        z = jnp.zeros((2 * _B, _H), jnp.float32)
        jax.lax.fori_loop(0, _L, step, (z, z), unroll=8)

    bilayer(p0f_ref, p0b_ref, dwh0f, dwh0b, ysd0_ref)

    ys0 = ysd0_ref[...].reshape(_L * _B, 2 * _H)
    p1f_ref[...] = (_dot(ys0, dwx1f[...]) + db1f[...]).reshape(_L, _B, 4 * _H)
    p1b_ref[...] = (_dot(ys0, dwx1b[...]) + db1b[...]).reshape(_L, _B, 4 * _H)

    bilayer(p1f_ref, p1b_ref, dwh1f, dwh1b, ysd1_ref)

    ys = ysd1_ref[...].reshape(_L * _B, 2 * _H)
    out_ref[...] = _dot(ys, wh[...]) + bh[...]


def kernel(sent_emb, params, sent_lengths):
    p = params

    scale = jnp.concatenate([jnp.full((2 * _H,), 0.5, jnp.float32),
                             jnp.ones((_H,), jnp.float32),
                             jnp.full((_H,), 0.5, jnp.float32)])

    def w(prefix):
        return (p[prefix + 'Wi'].T * scale, p[prefix + 'Wh'].T * scale,
                p[prefix + 'b'][None] * scale)

    def wcat(prefix):
        wx, wh, b = w(prefix)
        return jnp.concatenate([wx, wh], axis=0), b

    lens = sent_lengths.astype(jnp.float32)[:, None]

    sw = [x for pre in ('se0f_', 'se0b_', 'se1f_', 'se1b_')
          for x in wcat(pre)]
    dw = [x for pre in ('dl0f_', 'dl0b_', 'dl1f_', 'dl1b_') for x in w(pre)]

    vmem = pl.BlockSpec(memory_space=pltpu.MemorySpace.VMEM)
    logits = pl.pallas_call(
        _kernel,
        in_specs=[pl.BlockSpec(memory_space=pltpu.MemorySpace.HBM)]
        + [vmem] * (1 + len(sw) + len(dw) + 2),
        out_shape=jax.ShapeDtypeStruct((_L * _B, 2), jnp.float32),
        scratch_shapes=[pltpu.VMEM((_T, _S, _D), jnp.float32),
                        pltpu.VMEM((_T, _S, 2 * _H), jnp.float32)]
        + [pltpu.VMEM((_L, _B, 4 * _H), jnp.float32)] * 4
        + [pltpu.VMEM((_L, _B, 2 * _H), jnp.float32)] * 2
        + [pltpu.SemaphoreType.DMA((_T,))],
    )(sent_emb, lens, *sw, *dw, p['h2s_W'].T, p['h2s_b'][None])

    out = logits.reshape(_L, _B, 2).transpose(1, 0, 2)
    return out[:, :_L - 1].reshape((_L - 1) * _B, 2)


# speedup vs baseline: 1.3033x; 1.0205x over previous
import jax
import jax.numpy as jnp
from jax.experimental import pallas as pl
from jax.experimental.pallas import tpu as pltpu

_T = 32
_S = 512
_D = 128
_H = 128
_L = 64
_B = 8


def _gates(z, c):
    ti = jnp.tanh(z[:, 0:_H])
    tf = jnp.tanh(z[:, _H:2 * _H])
    tg = jnp.tanh(z[:, 2 * _H:3 * _H])
    to = jnp.tanh(z[:, 3 * _H:4 * _H])
    c_new = 0.5 * ((tf + 1.0) * c + (ti + 1.0) * tg)
    h_new = (0.5 * (to + 1.0)) * jnp.tanh(c_new)
    return h_new, c_new


def _dot(a, w):
    return jnp.dot(a, w, preferred_element_type=jnp.float32)


def _ld(ref, t):
    return ref[pl.ds(t, 1)][0]


def _kernel(x_hbm, len_ref,
            w0f, b0f, w0b, b0b, w1f, b1f, w1b, b1b,
            dwx0f, dwh0f, db0f, dwx0b, dwh0b, db0b,
            dwx1f, dwh1f, db1f, dwx1b, dwh1b, db1b,
            wh, bh, out_ref,
            xt_ref, ys0_ref, p0f_ref, p0b_ref, p1f_ref, p1b_ref,
            ysd0_ref, ysd1_ref, sems):
    ln = len_ref[...]

    def dma(t):
        return pltpu.make_async_copy(x_hbm.at[:, t], xt_ref.at[t], sems.at[t])

    for t in range(_T):
        dma(t).start()

    w0fv, b0fv = w0f[...], b0f[...]
    w0bv, b0bv = w0b[...], b0b[...]

    def cell(z_pre, h, c, m):
        hn, cn = _gates(z_pre, c)
        return jnp.where(m, hn, h), jnp.where(m, cn, c)

    zz = jnp.zeros((_S, _H), jnp.float32)
    hf = cf = hb = cb = zz
    for k in range(_T):
        t2 = _T - 1 - k
        if k < _T // 2:
            dma(k).wait()
            dma(t2).wait()
        m = ln > jnp.float32(k)
        m2 = ln > jnp.float32(t2)
        zf = _dot(jnp.concatenate([xt_ref[k], hf], axis=1), w0fv) + b0fv
        zb = _dot(jnp.concatenate([xt_ref[t2], hb], axis=1), w0bv) + b0bv
        hf, cf = cell(zf, hf, cf, m)
        hb, cb = cell(zb, hb, cb, m2)
        ys0_ref[k, :, 0:_H] = hf
        ys0_ref[t2, :, _H:2 * _H] = hb
    h0f, h0b = hf, hb

    w1fv, b1fv = w1f[...], b1f[...]
    w1bv, b1bv = w1b[...], b1b[...]

    hf = cf = hb = cb = zz
    for k in range(_T):
        t2 = _T - 1 - k
        m = ln > jnp.float32(k)
        m2 = ln > jnp.float32(t2)
        zf = _dot(jnp.concatenate([ys0_ref[k], hf], axis=1), w1fv) + b1fv
        zb = _dot(jnp.concatenate([ys0_ref[t2], hb], axis=1), w1bv) + b1bv
        hf, cf = cell(zf, hf, cf, m)
        hb, cb = cell(zb, hb, cb, m2)
    h1f, h1b = hf, hb

    enc = jnp.concatenate([h0f, h0b, h1f, h1b], axis=1)
    dxf = jnp.swapaxes(enc.reshape(_B, _L, 4 * _H), 0, 1).reshape(
        _L * _B, 4 * _H)

    p0f_ref[...] = (_dot(dxf, dwx0f[...]) + db0f[...]).reshape(_L, _B, 4 * _H)
    p0b_ref[...] = (_dot(dxf, dwx0b[...]) + db0b[...]).reshape(_L, _B, 4 * _H)

    def bilayer(pf_ref, pb_ref, whf, whb, ys_ref):
        whfv, whbv = whf[...], whb[...]
        h2 = c2 = jnp.zeros((2 * _B, _H), jnp.float32)
        for k in range(_L):
            t2 = _L - 1 - k
            zf = pf_ref[k] + _dot(h2[0:_B], whfv)
            zb = pb_ref[t2] + _dot(h2[_B:], whbv)
            h2, c2 = _gates(jnp.concatenate([zf, zb], axis=0), c2)
            ys_ref[k, :, 0:_H] = h2[0:_B]
            ys_ref[t2, :, _H:2 * _H] = h2[_B:]

    bilayer(p0f_ref, p0b_ref, dwh0f, dwh0b, ysd0_ref)

    ys0 = ysd0_ref[...].reshape(_L * _B, 2 * _H)
    p1f_ref[...] = (_dot(ys0, dwx1f[...]) + db1f[...]).reshape(_L, _B, 4 * _H)
    p1b_ref[...] = (_dot(ys0, dwx1b[...]) + db1b[...]).reshape(_L, _B, 4 * _H)

    bilayer(p1f_ref, p1b_ref, dwh1f, dwh1b, ysd1_ref)

    ys = ysd1_ref[...].reshape(_L * _B, 2 * _H)
    out_ref[...] = _dot(ys, wh[...]) + bh[...]


def kernel(sent_emb, params, sent_lengths):
    p = params

    scale = jnp.concatenate([jnp.full((2 * _H,), 0.5, jnp.float32),
                             jnp.ones((_H,), jnp.float32),
                             jnp.full((_H,), 0.5, jnp.float32)])

    def w(prefix):
        return (p[prefix + 'Wi'].T * scale, p[prefix + 'Wh'].T * scale,
                p[prefix + 'b'][None] * scale)

    def wcat(prefix):
        wx, wh, b = w(prefix)
        return jnp.concatenate([wx, wh], axis=0), b

    lens = sent_lengths.astype(jnp.float32)[:, None]

    sw = [x for pre in ('se0f_', 'se0b_', 'se1f_', 'se1b_')
          for x in wcat(pre)]
    dw = [x for pre in ('dl0f_', 'dl0b_', 'dl1f_', 'dl1b_') for x in w(pre)]

    vmem = pl.BlockSpec(memory_space=pltpu.MemorySpace.VMEM)
    logits = pl.pallas_call(
        _kernel,
        in_specs=[pl.BlockSpec(memory_space=pltpu.MemorySpace.HBM)]
        + [vmem] * (1 + len(sw) + len(dw) + 2),
        out_shape=jax.ShapeDtypeStruct((_L * _B, 2), jnp.float32),
        scratch_shapes=[pltpu.VMEM((_T, _S, _D), jnp.float32),
                        pltpu.VMEM((_T, _S, 2 * _H), jnp.float32)]
        + [pltpu.VMEM((_L, _B, 4 * _H), jnp.float32)] * 4
        + [pltpu.VMEM((_L, _B, 2 * _H), jnp.float32)] * 2
        + [pltpu.SemaphoreType.DMA((_T,))],
    )(sent_emb, lens, *sw, *dw, p['h2s_W'].T, p['h2s_b'][None])

    out = logits.reshape(_L, _B, 2).transpose(1, 0, 2)
    return out[:, :_L - 1].reshape((_L - 1) * _B, 2)
